# baseline (speedup 1.0000x reference)
import functools

import jax
import jax.numpy as jnp
from jax import lax
from jax.experimental import pallas as pl
from jax.experimental.pallas import tpu as pltpu
from jax.experimental.pallas import tpu_sc as plsc

_SLOTS = 9
_VEC = 16
_MAXE = 1024
_RING = 8


def kernel(inputs, embedding):
    if inputs.shape[-1] == 1:
        inputs = jnp.squeeze(inputs, axis=-1)
    (B,) = inputs.shape
    V, D = embedding.shape

    info = plsc.get_sparse_core_info()
    num_cores = info.num_cores
    nw = num_cores * info.num_subcores

    nb_total = (V + 127) // 128
    v_main = (V // 128) * 128
    n_blk_main = v_main // 128
    max_lb = (nb_total + nw - 1) // nw + 1
    n_lvec = (B + _VEC - 1) // _VEC
    n_rvec = (_MAXE + 2 * _VEC) // _VEC
    trash = n_rvec * _VEC
    sentinel = 255 << 21

    idx = inputs.astype(jnp.int32)
    table_t = embedding.T
    tail_rows = embedding[v_main:]

    @functools.partial(
        pl.kernel,
        mesh=plsc.VectorSubcoreMesh(core_axis_name="c", subcore_axis_name="s"),
        out_type=jax.ShapeDtypeStruct((B, D), jnp.float32),
        scratch_types=[
            pltpu.VMEM((B,), jnp.int32),
            pltpu.VMEM((n_rvec * _VEC + _VEC,), jnp.int32),
            pltpu.VMEM((_SLOTS, D, 128), jnp.float32),
            pltpu.VMEM((_RING, D), jnp.float32),
            pltpu.SMEM((256,), jnp.int32),
            pltpu.SMEM((256,), jnp.int32),
            pltpu.SMEM((_MAXE,), jnp.int32),
        ]
        + [pltpu.SemaphoreType.DMA] * (_SLOTS + _RING),
        compiler_params=pltpu.CompilerParams(needs_layout_passes=False),
    )
    def gather_kernel(idx_hbm, table_hbm, tail_hbm, out_hbm, idx_all, rlist,
                      blocks, rowring, cur, blist, grouped, *sems):
        out_sems = sems[_SLOTS:]
        wid = lax.axis_index("s") * num_cores + lax.axis_index("c")
        blk_lo = wid * nb_total // nw
        blk_hi = (wid + 1) * nb_total // nw
        nb_local = blk_hi - blk_lo

        pltpu.sync_copy(idx_hbm, idx_all)

        sent_vec = jnp.full((_VEC,), sentinel, jnp.int32)
        for t in range(n_rvec):
            rlist[pl.ds(t * _VEC, _VEC)] = sent_vec

        def filt(j2, cursor):
            for h in range(2):
                j = j2 * 2 + h
                v = idx_all[pl.ds(j * _VEC, _VEC)]
                ih = lax.shift_right_logical(v, 7)
                msk = jnp.logical_and(ih >= blk_lo, ih < blk_hi)
                entry = (
                    lax.shift_left(ih - blk_lo, 21)
                    | lax.shift_left(j * _VEC + lax.iota(jnp.int32, _VEC), 7)
                    | jnp.bitwise_and(v, 127)
                )
                mi = msk.astype(jnp.int32)
                pc = plsc.cumsum(mi)
                dest = jnp.where(msk, cursor + pc - 1, trash)
                plsc.store_scatter(rlist, [dest], entry)
                cursor = cursor + pc[_VEC - 1]
            return cursor

        n_real = lax.fori_loop(0, n_lvec // 2, filt, jnp.int32(0))
        n_scan = lax.shift_right_logical(n_real + _VEC - 1, 4) + 1

        def zero(b, _):
            cur[b] = jnp.int32(0)
            return ()

        lax.fori_loop(0, 256, zero, ())

        def hist(t, _):
            v = rlist[pl.ds(t * _VEC, _VEC)]
            for u in range(_VEC):
                lb = lax.shift_right_logical(v[u], 21)
                cur[lb] = cur[lb] + 1
            return ()

        lax.fori_loop(0, n_scan, hist, ())

        def scan(b, carry):
            run, m = carry
            c = cur[b]
            cur[b] = run

            @pl.when(c > 0)
            def _():
                blist[m] = lax.shift_left(b, 16) | run

            return run + c, m + (c > 0).astype(jnp.int32)

        run_m = lax.fori_loop(0, nb_local, scan, (jnp.int32(0), jnp.int32(0)))
        n_grp, m_blocks = run_m
        blist[m_blocks] = lax.shift_left(jnp.int32(255), 16) | n_grp

        def fire(j_scalar, slot):
            info_j = blist[j_scalar]
            ih = blk_lo + lax.shift_right_logical(info_j, 16)
            ih = jnp.minimum(ih, n_blk_main - 1)
            off = pl.multiple_of(ih * 128, 128)
            pltpu.async_copy(
                table_hbm.at[:, pl.ds(off, 128)], blocks.at[slot], sems[slot]
            )

        for u in range(_SLOTS):
            @pl.when(u < m_blocks)
            def _():
                fire(jnp.int32(u), u)

        def emit(t, _):
            v = rlist[pl.ds(t * _VEC, _VEC)]
            for u in range(_VEC):
                e = v[u]
                lb = lax.shift_right_logical(e, 21)

                @pl.when(lb < nb_local)
                def _():
                    d = cur[lb]
                    cur[lb] = d + 1
                    grouped[d] = jnp.bitwise_and(e, (1 << 21) - 1)

            return ()

        lax.fori_loop(0, n_scan, emit, ())

        def block_cycle(t, _):
            for u in range(_SLOTS):
                j = t * _SLOTS + u

                @pl.when(j < m_blocks)
                def _():
                    pltpu.make_async_copy(
                        table_hbm.at[:, pl.ds(0, 128)], blocks.at[u], sems[u]
                    ).wait()
                    info_j = blist[j]
                    lb = lax.shift_right_logical(info_j, 16)
                    start = jnp.bitwise_and(info_j, 0xFFFF)
                    end = jnp.bitwise_and(blist[j + 1], 0xFFFF)
                    ih = blk_lo + lb
                    is_tail = ih >= n_blk_main

                    def entry_chunk(c, _):
                        for r in range(_RING):
                            e = c * _RING + r

                            @pl.when(jnp.logical_and(e >= start, e < end))
                            def _():
                                ent = grouped[e]
                                il = jnp.bitwise_and(ent, 127)
                                pos = lax.shift_right_logical(ent, 7)

                                @pl.when(e >= _RING)
                                def _():
                                    pltpu.make_async_copy(
                                        out_hbm.at[0], rowring.at[r],
                                        out_sems[r],
                                    ).wait()

                                i2 = jnp.broadcast_to(il, (_VEC,))
                                for k in range(D // _VEC):
                                    i0 = jnp.full((_VEC,), u, jnp.int32)
                                    i1 = k * _VEC + lax.iota(jnp.int32, _VEC)
                                    vals = plsc.load_gather(blocks, [i0, i1, i2])
                                    rowring[r, pl.ds(k * _VEC, _VEC)] = vals

                                if V % 128 != 0:
                                    @pl.when(is_tail)
                                    def _():
                                        pltpu.sync_copy(
                                            tail_hbm.at[il], rowring.at[r]
                                        )

                                pltpu.async_copy(
                                    rowring.at[r], out_hbm.at[pos], out_sems[r]
                                )

                        return ()

                    lax.fori_loop(
                        lax.shift_right_logical(start, 3),
                        lax.shift_right_logical(end + _RING - 1, 3),
                        entry_chunk,
                        (),
                    )

                    @pl.when(j + _SLOTS < m_blocks)
                    def _():
                        fire(j + _SLOTS, u)

            return ()

        lax.fori_loop(0, (256 + _SLOTS - 1) // _SLOTS, block_cycle, ())

        for r in range(_RING):
            @pl.when(r < n_grp)
            def _():
                pltpu.make_async_copy(
                    out_hbm.at[0], rowring.at[r], out_sems[r]
                ).wait()

    return gather_kernel(idx, table_t, tail_rows)


# speedup vs baseline: 1.6839x; 1.6839x over previous
import functools

import jax
import jax.numpy as jnp
from jax import lax
from jax.experimental import pallas as pl
from jax.experimental.pallas import tpu as pltpu
from jax.experimental.pallas import tpu_sc as plsc

_SLOTS = 9
_VEC = 16
_MAXE = 1024
_RING = 8


def kernel(inputs, embedding):
    if inputs.shape[-1] == 1:
        inputs = jnp.squeeze(inputs, axis=-1)
    (B,) = inputs.shape
    V, D = embedding.shape

    info = plsc.get_sparse_core_info()
    num_cores = info.num_cores
    nw = num_cores * info.num_subcores

    nb_total = (V + 127) // 128
    v_main = (V // 128) * 128
    n_blk_main = v_main // 128
    max_lb = (nb_total + nw - 1) // nw + 1
    n_lvec = (B + _VEC - 1) // _VEC
    n_rvec = (_MAXE + 2 * _VEC) // _VEC
    trash = n_rvec * _VEC
    sentinel = 255 << 21

    idx = inputs.astype(jnp.int32)
    table_t = embedding.T
    tail_rows = embedding[v_main:]

    @functools.partial(
        pl.kernel,
        mesh=plsc.VectorSubcoreMesh(core_axis_name="c", subcore_axis_name="s"),
        out_type=jax.ShapeDtypeStruct((B, D), jnp.float32),
        scratch_types=[
            pltpu.VMEM((B,), jnp.int32),
            pltpu.VMEM((n_rvec * _VEC + _VEC,), jnp.int32),
            pltpu.VMEM((_SLOTS, D, 128), jnp.float32),
            pltpu.VMEM((_RING, D), jnp.float32),
            pltpu.SMEM((256,), jnp.int32),
            pltpu.SMEM((256,), jnp.int32),
            pltpu.SMEM((_MAXE,), jnp.int32),
        ]
        + [pltpu.SemaphoreType.DMA] * _SLOTS
        + [pltpu.SemaphoreType.DMA((_RING,))],
        compiler_params=pltpu.CompilerParams(needs_layout_passes=False),
    )
    def gather_kernel(idx_hbm, table_hbm, tail_hbm, out_hbm, idx_all, rlist,
                      blocks, rowring, cur, blist, grouped, *sems):
        out_sems = sems[_SLOTS]
        wid = lax.axis_index("s") * num_cores + lax.axis_index("c")
        blk_lo = wid * nb_total // nw
        blk_hi = (wid + 1) * nb_total // nw
        nb_local = blk_hi - blk_lo

        pltpu.sync_copy(idx_hbm, idx_all)

        sent_vec = jnp.full((_VEC,), sentinel, jnp.int32)
        for t in range(n_rvec):
            rlist[pl.ds(t * _VEC, _VEC)] = sent_vec

        def filt(j2, cursor):
            for h in range(2):
                j = j2 * 2 + h
                v = idx_all[pl.ds(j * _VEC, _VEC)]
                ih = lax.shift_right_logical(v, 7)
                msk = jnp.logical_and(ih >= blk_lo, ih < blk_hi)
                entry = (
                    lax.shift_left(ih - blk_lo, 21)
                    | lax.shift_left(j * _VEC + lax.iota(jnp.int32, _VEC), 7)
                    | jnp.bitwise_and(v, 127)
                )
                mi = msk.astype(jnp.int32)
                pc = plsc.cumsum(mi)
                dest = jnp.where(msk, cursor + pc - 1, trash)
                plsc.store_scatter(rlist, [dest], entry)
                cursor = cursor + pc[_VEC - 1]
            return cursor

        n_real = lax.fori_loop(0, n_lvec // 2, filt, jnp.int32(0))
        n_scan = lax.shift_right_logical(n_real + _VEC - 1, 4) + 1

        def zero(b, _):
            cur[b] = jnp.int32(0)
            return ()

        lax.fori_loop(0, 256, zero, ())

        def hist(t, _):
            v = rlist[pl.ds(t * _VEC, _VEC)]
            for u in range(_VEC):
                lb = lax.shift_right_logical(v[u], 21)
                cur[lb] = cur[lb] + 1
            return ()

        lax.fori_loop(0, n_scan, hist, ())

        def scan(b, carry):
            run, m = carry
            c = cur[b]
            cur[b] = run

            @pl.when(c > 0)
            def _():
                blist[m] = lax.shift_left(b, 16) | run

            return run + c, m + (c > 0).astype(jnp.int32)

        run_m = lax.fori_loop(0, nb_local, scan, (jnp.int32(0), jnp.int32(0)))
        n_grp, m_blocks = run_m
        blist[m_blocks] = lax.shift_left(jnp.int32(255), 16) | n_grp

        def fire(j_scalar, slot):
            info_j = blist[j_scalar]
            ih = blk_lo + lax.shift_right_logical(info_j, 16)
            ih = jnp.minimum(ih, n_blk_main - 1)
            off = pl.multiple_of(ih * 128, 128)
            pltpu.async_copy(
                table_hbm.at[:, pl.ds(off, 128)], blocks.at[slot], sems[slot]
            )

        for u in range(_SLOTS):
            @pl.when(u < m_blocks)
            def _():
                fire(jnp.int32(u), u)

        def emit(t, _):
            v = rlist[pl.ds(t * _VEC, _VEC)]
            for u in range(_VEC):
                e = v[u]
                lb = lax.shift_right_logical(e, 21)

                @pl.when(lb < nb_local)
                def _():
                    d = cur[lb]
                    cur[lb] = d + 1
                    grouped[d] = jnp.bitwise_and(e, (1 << 21) - 1)

            return ()

        lax.fori_loop(0, n_scan, emit, ())

        def block_cycle(t, _):
            for u in range(_SLOTS):
                j = t * _SLOTS + u

                @pl.when(j < m_blocks)
                def _():
                    pltpu.make_async_copy(
                        table_hbm.at[:, pl.ds(0, 128)], blocks.at[u], sems[u]
                    ).wait()
                    info_j = blist[j]
                    lb = lax.shift_right_logical(info_j, 16)
                    start = jnp.bitwise_and(info_j, 0xFFFF)
                    end = jnp.bitwise_and(blist[j + 1], 0xFFFF)
                    ih = blk_lo + lb
                    is_tail = ih >= n_blk_main

                    def entry_body(e, _):
                        ent = grouped[e]
                        il = jnp.bitwise_and(ent, 127)
                        pos = lax.shift_right_logical(ent, 7)
                        r = jnp.bitwise_and(e, _RING - 1)

                        @pl.when(e >= _RING)
                        def _():
                            pltpu.make_async_copy(
                                out_hbm.at[0], rowring.at[r], out_sems.at[r]
                            ).wait()

                        i2 = jnp.broadcast_to(il, (_VEC,))
                        for k in range(D // _VEC):
                            i0 = jnp.full((_VEC,), u, jnp.int32)
                            i1 = k * _VEC + lax.iota(jnp.int32, _VEC)
                            vals = plsc.load_gather(blocks, [i0, i1, i2])
                            rowring[r, pl.ds(k * _VEC, _VEC)] = vals

                        if V % 128 != 0:
                            @pl.when(is_tail)
                            def _():
                                pltpu.sync_copy(tail_hbm.at[il], rowring.at[r])

                        pltpu.async_copy(
                            rowring.at[r], out_hbm.at[pos], out_sems.at[r]
                        )
                        return ()

                    lax.fori_loop(start, end, entry_body, ())

                    @pl.when(j + _SLOTS < m_blocks)
                    def _():
                        fire(j + _SLOTS, u)

            return ()

        lax.fori_loop(0, (256 + _SLOTS - 1) // _SLOTS, block_cycle, ())

        def drain(i, _):
            r = jnp.bitwise_and(i, _RING - 1)
            pltpu.make_async_copy(
                out_hbm.at[0], rowring.at[r], out_sems.at[r]
            ).wait()
            return ()

        lax.fori_loop(
            jnp.maximum(n_grp - _RING, 0), n_grp, drain, ()
        )

    return gather_kernel(idx, table_t, tail_rows)
